# baseline (speedup 1.0000x reference)
import functools

import jax
import jax.numpy as jnp
from jax import lax
from jax.experimental import pallas as pl
from jax.experimental.pallas import tpu as pltpu
from jax.experimental.pallas import tpu_sc as plsc

_VOCAB = 100000
_BATCH = 16384
_HIST = 200
_TOTAL = _BATCH * _HIST

_NC = 2
_NS = 16
_NW = _NC * _NS
_L = 16

_VPAD = 100352
_IROWS = 512
_ICOLS = 6400
_ROWS_W = _IROWS // _NW
_CROWS = 1
_NCHUNK = _ROWS_W // _CROWS
_NVEC = _ICOLS // _L

_mesh = plsc.VectorSubcoreMesh(
    core_axis_name="c", subcore_axis_name="s", num_cores=_NC, num_subcores=_NS
)


@functools.partial(
    pl.kernel,
    out_type=jax.ShapeDtypeStruct((_NW * _VPAD,), jnp.int32),
    mesh=_mesh,
    scratch_types=[
        pltpu.VMEM((_VPAD,), jnp.int32),
        pltpu.VMEM((_CROWS, _ICOLS), jnp.int32),
        pltpu.VMEM((_CROWS, _ICOLS), jnp.int32),
        pltpu.VMEM((_CROWS, _ICOLS), jnp.int32),
        pltpu.SemaphoreType.DMA,
        pltpu.SemaphoreType.DMA,
        pltpu.SemaphoreType.DMA,
    ],
    compiler_params=pltpu.CompilerParams(needs_layout_passes=False),
)
def _sc_hist(ids_hbm, out_hbm, hist, buf0, buf1, buf2, sem0, sem1, sem2):
    wid = lax.axis_index("s") * _NC + lax.axis_index("c")
    bufs = (buf0, buf1, buf2)
    sems = (sem0, sem1, sem2)

    zero = jnp.zeros((_L,), jnp.int32)
    row0 = wid * _ROWS_W

    ones = jnp.ones((_L,), jnp.int32)

    def _bin_chunk(cur):
        @plsc.parallel_loop(0, _NVEC, step=1, unroll=8)
        def _scatter_body(j):
            ids = cur[0, pl.ds(j * _L, _L)]
            plsc.addupdate_scatter(hist, [ids], ones)

    _NBUF = 3

    def _chunk_copy(g, buf, sem):
        return pltpu.async_copy(
            ids_hbm.at[pl.ds(row0 + g * _CROWS, _CROWS), :], buf, sem
        )

    def _wait_chunk(p):
        pltpu.make_async_copy(
            ids_hbm.at[pl.ds(row0, _CROWS), :], bufs[p], sems[p]
        ).wait()

    for g in range(_NBUF - 1):
        _chunk_copy(g, bufs[g], sems[g])

    @plsc.parallel_loop(0, _VPAD, step=_L, unroll=8)
    def _zero_body(i):
        hist[pl.ds(i, _L)] = zero

    _FULL = _NCHUNK // _NBUF

    def _ring_body(h, hi):
        for p in range(_NBUF):
            g = _NBUF * h + p
            nxt = g + _NBUF - 1

            @pl.when(nxt < _NCHUNK)
            def _():
                _chunk_copy(nxt, bufs[(p + _NBUF - 1) % _NBUF],
                            sems[(p + _NBUF - 1) % _NBUF])

            _wait_chunk(p)
            _bin_chunk(bufs[p])
        return 0

    lax.fori_loop(0, _FULL, _ring_body, 0)
    for g in range(_FULL * _NBUF, _NCHUNK):
        p = g % _NBUF
        _wait_chunk(p)
        _bin_chunk(bufs[p])

    pltpu.sync_copy(hist, out_hbm.at[pl.ds(wid * _VPAD, _VPAD)])


def _tc_entropy_body(parts_ref, out_ref):
    counts = jnp.sum(parts_ref[...], axis=0)
    total = jnp.sum(counts)
    cf = counts.astype(jnp.float32)
    p = cf / total.astype(jnp.float32)
    safe_p = jnp.where(p > 0, p, 1.0)
    plogp = jnp.where(p > 0, p * jnp.log(safe_p), 0.0)
    out_ref[0, 0] = -jnp.sum(plogp)


_tc_entropy = pl.pallas_call(
    _tc_entropy_body,
    out_shape=jax.ShapeDtypeStruct((1, 1), jnp.float32),
    out_specs=pl.BlockSpec(memory_space=pltpu.SMEM),
)


def kernel(recommendations):
    partials = _sc_hist(recommendations.reshape(_IROWS, _ICOLS))
    ent = _tc_entropy(partials.reshape(_NW, _VPAD // 128, 128))
    return ent[0, 0]


# speedup vs baseline: 1.2912x; 1.2912x over previous
import functools

import jax
import jax.numpy as jnp
from jax import lax
from jax.experimental import pallas as pl
from jax.experimental.pallas import tpu as pltpu
from jax.experimental.pallas import tpu_sc as plsc

_VOCAB = 100000
_BATCH = 16384
_HIST = 200
_TOTAL = _BATCH * _HIST

_NC = 2
_NS = 16
_NW = _NC * _NS
_L = 16

_VPAD = 100352
_ROWS_W = _BATCH // _NW
_CROWS = 32
_NCHUNK = _ROWS_W // _CROWS
_NFULL = _HIST // _L
_TAIL = _HIST - _NFULL * _L

_mesh = plsc.VectorSubcoreMesh(
    core_axis_name="c", subcore_axis_name="s", num_cores=_NC, num_subcores=_NS
)


@functools.partial(
    pl.kernel,
    out_type=jax.ShapeDtypeStruct((_NW * _VPAD,), jnp.int32),
    mesh=_mesh,
    scratch_types=[
        pltpu.VMEM((_VPAD,), jnp.int32),
        pltpu.VMEM((_CROWS, _HIST), jnp.int32),
        pltpu.VMEM((_CROWS, _HIST), jnp.int32),
        pltpu.VMEM((_CROWS, _HIST), jnp.int32),
        pltpu.SemaphoreType.DMA,
        pltpu.SemaphoreType.DMA,
        pltpu.SemaphoreType.DMA,
    ],
    compiler_params=pltpu.CompilerParams(needs_layout_passes=False),
)
def _sc_hist(ids_hbm, out_hbm, hist, buf0, buf1, buf2, sem0, sem1, sem2):
    wid = lax.axis_index("s") * _NC + lax.axis_index("c")
    bufs = (buf0, buf1, buf2)
    sems = (sem0, sem1, sem2)

    zero = jnp.zeros((_L,), jnp.int32)
    row0 = wid * _ROWS_W
    tail_valid = lax.iota(jnp.int32, _L) >= (_L - _TAIL)

    ones = jnp.ones((_L,), jnp.int32)

    def _bin_chunk(cur):
        @plsc.parallel_loop(0, _CROWS, step=1, unroll=2)
        def _scatter_body(j):
            for k in range(_NFULL):
                ids = cur[j, pl.ds(k * _L, _L)]
                plsc.addupdate_scatter(hist, [ids], ones)
            ids = cur[j, pl.ds(_HIST - _L, _L)]
            plsc.addupdate_scatter(hist, [ids], ones, mask=tail_valid)

    _NBUF = 3

    def _chunk_copy(g, buf, sem):
        return pltpu.async_copy(
            ids_hbm.at[pl.ds(row0 + g * _CROWS, _CROWS), :], buf, sem
        )

    def _wait_chunk(p):
        pltpu.make_async_copy(
            ids_hbm.at[pl.ds(row0, _CROWS), :], bufs[p], sems[p]
        ).wait()

    for g in range(_NBUF - 1):
        _chunk_copy(g, bufs[g], sems[g])

    @plsc.parallel_loop(0, _VPAD, step=_L, unroll=8)
    def _zero_body(i):
        hist[pl.ds(i, _L)] = zero

    _FULL = _NCHUNK // _NBUF

    def _ring_body(h, hi):
        for p in range(_NBUF):
            g = _NBUF * h + p
            nxt = g + _NBUF - 1

            @pl.when(nxt < _NCHUNK)
            def _():
                _chunk_copy(nxt, bufs[(p + _NBUF - 1) % _NBUF],
                            sems[(p + _NBUF - 1) % _NBUF])

            _wait_chunk(p)
            _bin_chunk(bufs[p])
        return 0

    lax.fori_loop(0, _FULL, _ring_body, 0)
    for g in range(_FULL * _NBUF, _NCHUNK):
        p = g % _NBUF
        _wait_chunk(p)
        _bin_chunk(bufs[p])

    pltpu.sync_copy(hist, out_hbm.at[pl.ds(wid * _VPAD, _VPAD)])


_VROWS = _VPAD // 128
_GRID = 7
_VBLK = _VROWS // _GRID


def _tc_entropy_body(parts_ref, out_ref, acc_ref):
    g = pl.program_id(0)

    @pl.when(g == 0)
    def _():
        acc_ref[0] = 0.0

    counts = jnp.sum(parts_ref[...], axis=0)
    cf = counts.astype(jnp.float32)
    p = cf * (1.0 / float(_TOTAL))
    safe_p = jnp.where(p > 0, p, 1.0)
    plogp = jnp.where(p > 0, p * jnp.log(safe_p), 0.0)
    acc_ref[0] += jnp.sum(plogp)

    @pl.when(g == _GRID - 1)
    def _():
        out_ref[0, 0] = -acc_ref[0]


_tc_entropy = pl.pallas_call(
    _tc_entropy_body,
    grid=(_GRID,),
    in_specs=[pl.BlockSpec((_NW, _VBLK, 128), lambda g: (0, g, 0))],
    out_specs=pl.BlockSpec(memory_space=pltpu.SMEM),
    out_shape=jax.ShapeDtypeStruct((1, 1), jnp.float32),
    scratch_shapes=[pltpu.SMEM((1,), jnp.float32)],
)


def kernel(recommendations):
    partials = _sc_hist(recommendations)
    ent = _tc_entropy(partials.reshape(_NW, _VPAD // 128, 128))
    return ent[0, 0]


# speedup vs baseline: 1.3406x; 1.0383x over previous
import functools

import jax
import jax.numpy as jnp
from jax import lax
from jax.experimental import pallas as pl
from jax.experimental.pallas import tpu as pltpu
from jax.experimental.pallas import tpu_sc as plsc

_VOCAB = 100000
_BATCH = 16384
_HIST = 200
_TOTAL = _BATCH * _HIST

_NC = 2
_NS = 16
_NW = _NC * _NS
_L = 16

_VPAD = 100352
_ROWS_W = _BATCH // _NW
_CROWS = 32
_NCHUNK = _ROWS_W // _CROWS
_NFULL = _HIST // _L
_TAIL = _HIST - _NFULL * _L

_mesh = plsc.VectorSubcoreMesh(
    core_axis_name="c", subcore_axis_name="s", num_cores=_NC, num_subcores=_NS
)


@functools.partial(
    pl.kernel,
    out_type=jax.ShapeDtypeStruct((_NW * _VPAD,), jnp.int32),
    mesh=_mesh,
    scratch_types=[
        pltpu.VMEM((_VPAD,), jnp.int32),
        pltpu.VMEM((_CROWS, _HIST), jnp.int32),
        pltpu.VMEM((_CROWS, _HIST), jnp.int32),
        pltpu.VMEM((_CROWS, _HIST), jnp.int32),
        pltpu.SemaphoreType.DMA,
        pltpu.SemaphoreType.DMA,
        pltpu.SemaphoreType.DMA,
    ],
    compiler_params=pltpu.CompilerParams(needs_layout_passes=False),
)
def _sc_hist(ids_hbm, out_hbm, hist, buf0, buf1, buf2, sem0, sem1, sem2):
    wid = lax.axis_index("s") * _NC + lax.axis_index("c")
    bufs = (buf0, buf1, buf2)
    sems = (sem0, sem1, sem2)

    zero = jnp.zeros((_L,), jnp.int32)
    row0 = wid * _ROWS_W
    tail_valid = lax.iota(jnp.int32, _L) >= (_L - _TAIL)

    ones = jnp.ones((_L,), jnp.int32)

    def _bin_chunk(cur):
        @plsc.parallel_loop(0, _CROWS, step=1, unroll=2)
        def _scatter_body(j):
            for k in range(_NFULL):
                ids = cur[j, pl.ds(k * _L, _L)]
                plsc.addupdate_scatter(hist, [ids], ones)
            ids = cur[j, pl.ds(_HIST - _L, _L)]
            plsc.addupdate_scatter(hist, [ids], ones, mask=tail_valid)

    _NBUF = 3

    def _chunk_copy(g, buf, sem):
        return pltpu.async_copy(
            ids_hbm.at[pl.ds(row0 + g * _CROWS, _CROWS), :], buf, sem
        )

    def _wait_chunk(p):
        pltpu.make_async_copy(
            ids_hbm.at[pl.ds(row0, _CROWS), :], bufs[p], sems[p]
        ).wait()

    for g in range(_NBUF - 1):
        _chunk_copy(g, bufs[g], sems[g])

    @plsc.parallel_loop(0, _VPAD, step=_L, unroll=8)
    def _zero_body(i):
        hist[pl.ds(i, _L)] = zero

    _FULL = _NCHUNK // _NBUF

    def _ring_body(h, hi):
        for p in range(_NBUF):
            g = _NBUF * h + p
            nxt = g + _NBUF - 1

            @pl.when(nxt < _NCHUNK)
            def _():
                _chunk_copy(nxt, bufs[(p + _NBUF - 1) % _NBUF],
                            sems[(p + _NBUF - 1) % _NBUF])

            _wait_chunk(p)
            _bin_chunk(bufs[p])
        return 0

    lax.fori_loop(0, _FULL, _ring_body, 0)
    for g in range(_FULL * _NBUF, _NCHUNK):
        p = g % _NBUF
        _wait_chunk(p)
        _bin_chunk(bufs[p])

    pltpu.sync_copy(hist, out_hbm.at[pl.ds(wid * _VPAD, _VPAD)])


_VROWS = _VPAD // 128
_GRID = 2
_VBLK = _VROWS // _GRID


def _tc_entropy_body(parts_ref, out_ref, acc_ref):
    g = pl.program_id(0)

    @pl.when(g == 0)
    def _():
        acc_ref[0] = 0.0

    counts = jnp.sum(parts_ref[...], axis=0)
    cf = counts.astype(jnp.float32)
    p = cf * (1.0 / float(_TOTAL))
    safe_p = jnp.where(p > 0, p, 1.0)
    plogp = jnp.where(p > 0, p * jnp.log(safe_p), 0.0)
    acc_ref[0] += jnp.sum(plogp)

    @pl.when(g == _GRID - 1)
    def _():
        out_ref[0, 0] = -acc_ref[0]


_tc_entropy = pl.pallas_call(
    _tc_entropy_body,
    grid=(_GRID,),
    in_specs=[pl.BlockSpec((_NW, _VBLK, 128), lambda g: (0, g, 0))],
    out_specs=pl.BlockSpec(memory_space=pltpu.SMEM),
    out_shape=jax.ShapeDtypeStruct((1, 1), jnp.float32),
    scratch_shapes=[pltpu.SMEM((1,), jnp.float32)],
)


def kernel(recommendations):
    partials = _sc_hist(recommendations)
    ent = _tc_entropy(partials.reshape(_NW, _VPAD // 128, 128))
    return ent[0, 0]
